# baseline (speedup 1.0000x reference)
import functools

import jax
import jax.numpy as jnp
from jax import lax
from jax.experimental import pallas as pl
from jax.experimental.pallas import tpu as pltpu
from jax.experimental.pallas import tpu_sc as plsc

BATCH = 1024
CTX = 50
EMBED_DIM = 64
VOCAB = 100000
OUTPUT_SIZE = 100000

NUM_CORES = 2
NUM_SUBCORES = 16
NUM_WORKERS = NUM_CORES * NUM_SUBCORES
B_PER_W = BATCH // NUM_WORKERS
IDX_PER_W = B_PER_W * CTX
GATHER_CHUNK = 80
NUM_GATHERS = IDX_PER_W // GATHER_CHUNK
LANES = 16
COL_CHUNKS = EMBED_DIM // LANES


ROWS_HALF = IDX_PER_W // 2
ITEMS_HALF = B_PER_W // 2
PAIR_W = 2 * EMBED_DIM


def _pool_body(idx_hbm, table_hbm, out_hbm, idx_v, rows_v, acc_v, sem):
    wid = lax.axis_index("s") * NUM_CORES + lax.axis_index("c")
    base = wid * IDX_PER_W
    pltpu.sync_copy(idx_hbm.at[pl.ds(base, IDX_PER_W)], idx_v)
    copies = [
        pltpu.async_copy(
            table_hbm.at[idx_v.at[pl.ds(j * GATHER_CHUNK, GATHER_CHUNK)]],
            rows_v.at[pl.ds(j * GATHER_CHUNK, GATHER_CHUNK)],
            sem,
        )
        for j in range(NUM_GATHERS)
    ]
    for cp in copies:
        cp.wait()

    def body_b(b, carry):
        r0 = b * CTX
        accs = [rows_v[r0, pl.ds(k * LANES, LANES)] for k in range(COL_CHUNKS)]
        for c in range(1, CTX):
            for k in range(COL_CHUNKS):
                accs[k] = accs[k] + rows_v[r0 + c, pl.ds(k * LANES, LANES)]
        zero = jnp.zeros((LANES,), jnp.float32)
        for k in range(COL_CHUNKS):
            acc_v[b, pl.ds(k * LANES, LANES)] = accs[k]
            acc_v[b, pl.ds(EMBED_DIM + k * LANES, LANES)] = zero
        return carry

    lax.fori_loop(0, B_PER_W, body_b, 0)
    pltpu.sync_copy(acc_v, out_hbm.at[pl.ds(wid * B_PER_W, B_PER_W)])


@functools.cache
def _pool():
    return pl.kernel(
        _pool_body,
        out_type=jax.ShapeDtypeStruct((BATCH, PAIR_W), jnp.float32),
        mesh=plsc.VectorSubcoreMesh(core_axis_name="c", subcore_axis_name="s"),
        scratch_types=[
            pltpu.VMEM((IDX_PER_W,), jnp.int32),
            pltpu.VMEM((IDX_PER_W, EMBED_DIM), jnp.float32),
            pltpu.VMEM((B_PER_W, PAIR_W), jnp.float32),
            pltpu.SemaphoreType.DMA,
        ],
        compiler_params=pltpu.CompilerParams(use_tc_tiling_on_sc=False),
    )


BN = 2048


def _mm_body(w_ref, x_ref, b_ref, o_ref):
    o_ref[...] = (
        lax.dot_general(
            w_ref[...],
            x_ref[:, :EMBED_DIM],
            (((0,), (1,)), ((), ())),
            preferred_element_type=jnp.float32,
        )
        + jnp.transpose(b_ref[...], (1, 0))
    )


def _matmul_t(w_t, pooled, b2d):
    grid = (pl.cdiv(OUTPUT_SIZE, BN),)
    return pl.pallas_call(
        _mm_body,
        grid=grid,
        in_specs=[
            pl.BlockSpec((EMBED_DIM, BN), lambda i: (0, i)),
            pl.BlockSpec((BATCH, PAIR_W), lambda i: (0, 0)),
            pl.BlockSpec((1, BN), lambda i: (0, i)),
        ],
        out_specs=pl.BlockSpec((BN, BATCH), lambda i: (i, 0)),
        out_shape=jax.ShapeDtypeStruct((OUTPUT_SIZE, BATCH), jnp.float32),
    )(w_t, pooled, b2d)


def kernel(inputs, embed_table, W, b):
    idx_flat = inputs.astype(jnp.int32).reshape(-1)
    pooled128 = _pool()(idx_flat, embed_table)
    logits_t = _matmul_t(W.T, pooled128, b.reshape(1, OUTPUT_SIZE))
    return logits_t.T


# speedup vs baseline: 1.0088x; 1.0088x over previous
import functools

import jax
import jax.numpy as jnp
from jax import lax
from jax.experimental import pallas as pl
from jax.experimental.pallas import tpu as pltpu
from jax.experimental.pallas import tpu_sc as plsc

BATCH = 1024
CTX = 50
EMBED_DIM = 64
VOCAB = 100000
OUTPUT_SIZE = 100000

NUM_CORES = 2
NUM_SUBCORES = 16
NUM_WORKERS = NUM_CORES * NUM_SUBCORES
B_PER_W = BATCH // NUM_WORKERS
IDX_PER_W = B_PER_W * CTX
GATHER_CHUNK = 80
NUM_GATHERS = IDX_PER_W // GATHER_CHUNK
LANES = 16
COL_CHUNKS = EMBED_DIM // LANES


ROWS_HALF = IDX_PER_W // 2
ITEMS_HALF = B_PER_W // 2
PAIR_W = 2 * EMBED_DIM


def _pool_body(idx_hbm, table_hbm, out_hbm, idx_v, rows_v, acc_v, sem):
    wid = lax.axis_index("s") * NUM_CORES + lax.axis_index("c")
    base = wid * IDX_PER_W
    pltpu.sync_copy(idx_hbm.at[pl.ds(base, IDX_PER_W)], idx_v)
    copies = [
        pltpu.async_copy(
            table_hbm.at[idx_v.at[pl.ds(j * GATHER_CHUNK, GATHER_CHUNK)]],
            rows_v.at[pl.ds(j * GATHER_CHUNK, GATHER_CHUNK)],
            sem,
        )
        for j in range(NUM_GATHERS)
    ]
    for cp in copies:
        cp.wait()

    def body_b(b, carry):
        r0 = b * CTX
        accs = [rows_v[r0, pl.ds(k * LANES, LANES)] for k in range(COL_CHUNKS)]
        for c in range(1, CTX):
            for k in range(COL_CHUNKS):
                accs[k] = accs[k] + rows_v[r0 + c, pl.ds(k * LANES, LANES)]
        zero = jnp.zeros((LANES,), jnp.float32)
        for k in range(COL_CHUNKS):
            acc_v[b, pl.ds(k * LANES, LANES)] = accs[k]
            acc_v[b, pl.ds(EMBED_DIM + k * LANES, LANES)] = zero
        return carry

    lax.fori_loop(0, B_PER_W, body_b, 0)
    pltpu.sync_copy(acc_v, out_hbm.at[pl.ds(wid * B_PER_W, B_PER_W)])


@functools.cache
def _pool():
    return pl.kernel(
        _pool_body,
        out_type=jax.ShapeDtypeStruct((BATCH, PAIR_W), jnp.float32),
        mesh=plsc.VectorSubcoreMesh(core_axis_name="c", subcore_axis_name="s"),
        scratch_types=[
            pltpu.VMEM((IDX_PER_W,), jnp.int32),
            pltpu.VMEM((IDX_PER_W, EMBED_DIM), jnp.float32),
            pltpu.VMEM((B_PER_W, PAIR_W), jnp.float32),
            pltpu.SemaphoreType.DMA,
        ],
        compiler_params=pltpu.CompilerParams(use_tc_tiling_on_sc=False),
    )


BN = 4096


def _mm_body(w_ref, x_ref, b_ref, o_ref):
    o_ref[...] = (
        lax.dot_general(
            w_ref[...],
            x_ref[:, :EMBED_DIM],
            (((0,), (1,)), ((), ())),
            preferred_element_type=jnp.float32,
        )
        + jnp.transpose(b_ref[...], (1, 0))
    )


def _matmul_t(w_t, pooled, b2d):
    grid = (pl.cdiv(OUTPUT_SIZE, BN),)
    return pl.pallas_call(
        _mm_body,
        grid=grid,
        in_specs=[
            pl.BlockSpec((EMBED_DIM, BN), lambda i: (0, i)),
            pl.BlockSpec((BATCH, PAIR_W), lambda i: (0, 0)),
            pl.BlockSpec((1, BN), lambda i: (0, i)),
        ],
        out_specs=pl.BlockSpec((BN, BATCH), lambda i: (i, 0)),
        out_shape=jax.ShapeDtypeStruct((OUTPUT_SIZE, BATCH), jnp.float32),
    )(w_t, pooled, b2d)


def kernel(inputs, embed_table, W, b):
    idx_flat = inputs.astype(jnp.int32).reshape(-1)
    pooled128 = _pool()(idx_flat, embed_table)
    logits_t = _matmul_t(W.T, pooled128, b.reshape(1, OUTPUT_SIZE))
    return logits_t.T
